# baseline (speedup 1.0000x reference)
import functools

import jax
import jax.numpy as jnp
from jax import lax
from jax.experimental import pallas as pl
from jax.experimental.pallas import tpu as pltpu
from jax.experimental.pallas import tpu_sc as plsc

B = 4096
NS = 26
V = 100000
D = 16
L = 50
NF = NS + 1
R = NF * D

NC = 2
NSUB = 16
NW = NC * NSUB
BW = B // NW
LH = L // 2
GD = BW // D


def _body(tab, seq, sidx, qidx, out, sidx_v, qidx_v, asm_v, srows_v, cnt_v,
          sem_a, sem_b):
    c = lax.axis_index("c")
    s = lax.axis_index("s")
    w = s * NC + c
    b0 = w * BW

    pltpu.sync_copy(sidx.at[:, pl.ds(b0, BW)], sidx_v)
    pltpu.sync_copy(qidx.at[:, pl.ds(b0, BW)], qidx_v)

    def sparse_f(f, carry):
        for d in range(D):
            pltpu.async_copy(tab.at[f * D + d].at[sidx_v.at[f]],
                             asm_v.at[f * D + d], sem_a)
        return carry

    lax.fori_loop(0, NS, sparse_f, 0)

    def seq_issue(l, off):
        for d in range(D):
            pltpu.async_copy(seq.at[d].at[qidx_v.at[off + l]],
                             srows_v.at[d, pl.ds(l * BW, BW)], sem_b)
        return off

    def drain_seq_half():
        pltpu.make_async_copy(seq.at[:, pl.ds(0, LH * BW)], srows_v,
                              sem_b).wait()

    def clear_acc(k, carry):
        d = k // GD
        g16 = (k % GD) * D
        asm_v[NS * D + d, pl.ds(g16, D)] = jnp.zeros((D,), jnp.float32)
        cnt_v[d, pl.ds(g16, D)] = jnp.zeros((D,), jnp.float32)
        return carry

    def seq_accum(k, carry):
        d = k // GD
        g16 = (k % GD) * D
        acc = asm_v[NS * D + d, pl.ds(g16, D)]
        cnt = cnt_v[d, pl.ds(g16, D)]
        for l in range(LH):
            v = srows_v[d, pl.ds(l * BW + g16, D)]
            acc = acc + v
            cnt = cnt + jnp.where(v != 0.0, 1.0, 0.0)
        asm_v[NS * D + d, pl.ds(g16, D)] = acc
        cnt_v[d, pl.ds(g16, D)] = cnt
        return carry

    def divide(k, carry):
        d = k // GD
        g16 = (k % GD) * D
        acc = asm_v[NS * D + d, pl.ds(g16, D)]
        cnt = cnt_v[d, pl.ds(g16, D)]
        asm_v[NS * D + d, pl.ds(g16, D)] = acc / (cnt + 1e-16)
        return carry

    lax.fori_loop(0, BW, clear_acc, 0)
    lax.fori_loop(0, LH, seq_issue, 0)
    drain_seq_half()
    lax.fori_loop(0, BW, seq_accum, 0)
    lax.fori_loop(0, LH, seq_issue, LH)
    pltpu.make_async_copy(tab.at[:, pl.ds(0, BW)],
                          asm_v.at[pl.ds(0, NS * D)], sem_a).wait()
    drain_seq_half()
    lax.fori_loop(0, BW, seq_accum, 0)
    lax.fori_loop(0, BW, divide, 0)

    pltpu.sync_copy(asm_v, out.at[:, pl.ds(b0, BW)])


_sc_call = functools.partial(
    pl.kernel,
    out_type=jax.ShapeDtypeStruct((R, B), jnp.float32),
    mesh=plsc.VectorSubcoreMesh(core_axis_name="c", subcore_axis_name="s"),
    compiler_params=pltpu.CompilerParams(use_tc_tiling_on_sc=False,
                                         skip_device_barrier=True,
                                         disable_bounds_checks=True,
                                         disable_semaphore_checks=True),
    scratch_types=[
        pltpu.VMEM((NS, BW), jnp.int32),
        pltpu.VMEM((L, BW), jnp.int32),
        pltpu.VMEM((R, BW), jnp.float32),
        pltpu.VMEM((D, LH * BW), jnp.float32),
        pltpu.VMEM((D, BW), jnp.float32),
        pltpu.SemaphoreType.DMA,
        pltpu.SemaphoreType.DMA,
    ],
)(_body)


@jax.jit
def kernel(sparse_indices, seq_indices, sparse_tables, seq_table):
    sidxT = jnp.transpose(sparse_indices).astype(jnp.int32)
    qidxT = jnp.transpose(seq_indices).astype(jnp.int32)
    tabT = jnp.transpose(sparse_tables, (0, 2, 1)).reshape(NS * D, V)
    seqT = jnp.transpose(seq_table)

    outT = _sc_call(tabT, seqT, sidxT, qidxT)
    return jnp.transpose(outT.reshape(NF, D, B), (2, 0, 1))


# speedup vs baseline: 1.0017x; 1.0017x over previous
import functools

import jax
import jax.numpy as jnp
from jax import lax
from jax.experimental import pallas as pl
from jax.experimental.pallas import tpu as pltpu
from jax.experimental.pallas import tpu_sc as plsc

B = 4096
NS = 26
V = 100000
D = 16
L = 50
NF = NS + 1
R = NF * D

NC = 2
NSUB = 16
NW = NC * NSUB
BW = B // NW
LH = L // 2
GD = BW // D


def _body(tab, seq, sidx, qidx, out, sidx_v, qidx_v, asm_v, srows_v, cnt_v,
          sem_a, sem_b):
    c = lax.axis_index("c")
    s = lax.axis_index("s")
    w = s * NC + c
    b0 = w * BW

    h_sidx = pltpu.async_copy(sidx.at[:, pl.ds(b0, BW)], sidx_v, sem_a)
    h_qidx = pltpu.async_copy(qidx.at[:, pl.ds(b0, BW)], qidx_v, sem_b)
    h_sidx.wait()

    def sparse_f(f, carry):
        for d in range(D):
            pltpu.async_copy(tab.at[f * D + d].at[sidx_v.at[f]],
                             asm_v.at[f * D + d], sem_a)
        return carry

    lax.fori_loop(0, NS, sparse_f, 0)
    h_qidx.wait()

    def seq_issue(l, off):
        for d in range(D):
            pltpu.async_copy(seq.at[d].at[qidx_v.at[off + l]],
                             srows_v.at[d, pl.ds(l * BW, BW)], sem_b)
        return off

    def drain_seq_half():
        pltpu.make_async_copy(seq.at[:, pl.ds(0, LH * BW)], srows_v,
                              sem_b).wait()

    def clear_acc(k, carry):
        d = k // GD
        g16 = (k % GD) * D
        asm_v[NS * D + d, pl.ds(g16, D)] = jnp.zeros((D,), jnp.float32)
        cnt_v[d, pl.ds(g16, D)] = jnp.zeros((D,), jnp.float32)
        return carry

    def seq_accum(k, carry):
        d = k // GD
        g16 = (k % GD) * D
        acc = asm_v[NS * D + d, pl.ds(g16, D)]
        cnt = cnt_v[d, pl.ds(g16, D)]
        for l in range(LH):
            v = srows_v[d, pl.ds(l * BW + g16, D)]
            acc = acc + v
            cnt = cnt + jnp.where(v != 0.0, 1.0, 0.0)
        asm_v[NS * D + d, pl.ds(g16, D)] = acc
        cnt_v[d, pl.ds(g16, D)] = cnt
        return carry

    def seq_accum_final(k, carry):
        d = k // GD
        g16 = (k % GD) * D
        acc = asm_v[NS * D + d, pl.ds(g16, D)]
        cnt = cnt_v[d, pl.ds(g16, D)]
        for l in range(LH):
            v = srows_v[d, pl.ds(l * BW + g16, D)]
            acc = acc + v
            cnt = cnt + jnp.where(v != 0.0, 1.0, 0.0)
        asm_v[NS * D + d, pl.ds(g16, D)] = acc / (cnt + 1e-16)
        return carry

    lax.fori_loop(0, LH, seq_issue, 0)
    lax.fori_loop(0, BW, clear_acc, 0)
    drain_seq_half()
    lax.fori_loop(0, BW, seq_accum, 0)
    lax.fori_loop(0, LH, seq_issue, LH)
    pltpu.make_async_copy(tab.at[:, pl.ds(0, BW)],
                          asm_v.at[pl.ds(0, NS * D)], sem_a).wait()
    h_out = pltpu.async_copy(asm_v.at[pl.ds(0, NS * D)],
                             out.at[pl.ds(0, NS * D), pl.ds(b0, BW)], sem_a)
    drain_seq_half()
    lax.fori_loop(0, BW, seq_accum_final, 0)
    h_pool = pltpu.async_copy(asm_v.at[pl.ds(NS * D, D)],
                              out.at[pl.ds(NS * D, D), pl.ds(b0, BW)], sem_a)
    h_out.wait()
    h_pool.wait()


_sc_call = functools.partial(
    pl.kernel,
    out_type=jax.ShapeDtypeStruct((R, B), jnp.float32),
    mesh=plsc.VectorSubcoreMesh(core_axis_name="c", subcore_axis_name="s"),
    compiler_params=pltpu.CompilerParams(use_tc_tiling_on_sc=False,
                                         skip_device_barrier=True,
                                         disable_bounds_checks=True,
                                         disable_semaphore_checks=True),
    scratch_types=[
        pltpu.VMEM((NS, BW), jnp.int32),
        pltpu.VMEM((L, BW), jnp.int32),
        pltpu.VMEM((R, BW), jnp.float32),
        pltpu.VMEM((D, LH * BW), jnp.float32),
        pltpu.VMEM((D, BW), jnp.float32),
        pltpu.SemaphoreType.DMA,
        pltpu.SemaphoreType.DMA,
    ],
)(_body)


@jax.jit
def kernel(sparse_indices, seq_indices, sparse_tables, seq_table):
    sidxT = jnp.transpose(sparse_indices).astype(jnp.int32)
    qidxT = jnp.transpose(seq_indices).astype(jnp.int32)
    tabT = jnp.transpose(sparse_tables, (0, 2, 1)).reshape(NS * D, V)
    seqT = jnp.transpose(seq_table)

    outT = _sc_call(tabT, seqT, sidxT, qidxT)
    return jnp.transpose(outT.reshape(NF, D, B), (2, 0, 1))


# speedup vs baseline: 1.0101x; 1.0084x over previous
import functools

import jax
import jax.numpy as jnp
from jax import lax
from jax.experimental import pallas as pl
from jax.experimental.pallas import tpu as pltpu
from jax.experimental.pallas import tpu_sc as plsc

B = 4096
NS = 26
V = 100000
D = 16
L = 50
NF = NS + 1
R = NF * D

NC = 2
NSUB = 16
NW = NC * NSUB
BW = B // NW
QL = 13
QS = 12
GD = BW // D


def _body(tab, seq, sidx, qidx, out, sidx_v, qidx_v, asm_v, sq0_v, sq1_v,
          cnt_v, sem_a, sem_b):
    c = lax.axis_index("c")
    s = lax.axis_index("s")
    w = s * NC + c
    b0 = w * BW

    h_sidx = pltpu.async_copy(sidx.at[:, pl.ds(b0, BW)], sidx_v, sem_a)
    h_qidx = pltpu.async_copy(qidx.at[:, pl.ds(b0, BW)], qidx_v, sem_b)
    h_sidx.wait()

    def sparse_f(f, carry):
        for d in range(D):
            pltpu.async_copy(tab.at[f * D + d].at[sidx_v.at[f]],
                             asm_v.at[f * D + d], sem_a)
        return carry

    lax.fori_loop(0, NS, sparse_f, 0)
    h_qidx.wait()

    bufs = [sq0_v, sq1_v, sq0_v, sq1_v]
    offs = [0, QL, 2 * QL, 2 * QL + QS]
    qls = [QL, QL, QS, QS]

    def issue_q(q):
        buf, off = bufs[q], offs[q]

        def body(l, carry):
            for d in range(D):
                pltpu.async_copy(seq.at[d].at[qidx_v.at[off + l]],
                                 buf.at[d, pl.ds(l * BW, BW)], sem_b)
            return carry

        lax.fori_loop(0, qls[q], body, 0)

    def drain_q(q):
        n = qls[q] * BW
        pltpu.make_async_copy(seq.at[:, pl.ds(0, n)],
                              bufs[q].at[:, pl.ds(0, n)], sem_b).wait()

    def clear_acc(k, carry):
        d = k // GD
        g16 = (k % GD) * D
        asm_v[NS * D + d, pl.ds(g16, D)] = jnp.zeros((D,), jnp.float32)
        cnt_v[d, pl.ds(g16, D)] = jnp.zeros((D,), jnp.float32)
        return carry

    def accum_q(q, final=False):
        buf = bufs[q]

        def body(k, carry):
            d = k // GD
            g16 = (k % GD) * D
            acc = asm_v[NS * D + d, pl.ds(g16, D)]
            cnt = cnt_v[d, pl.ds(g16, D)]
            for l in range(qls[q]):
                v = buf[d, pl.ds(l * BW + g16, D)]
                acc = acc + v
                cnt = cnt + jnp.where(v != 0.0, 1.0, 0.0)
            if final:
                asm_v[NS * D + d, pl.ds(g16, D)] = acc / (cnt + 1e-16)
            else:
                asm_v[NS * D + d, pl.ds(g16, D)] = acc
                cnt_v[d, pl.ds(g16, D)] = cnt
            return carry

        lax.fori_loop(0, BW, body, 0)

    issue_q(0)
    issue_q(1)
    lax.fori_loop(0, BW, clear_acc, 0)
    drain_q(0)
    accum_q(0)
    issue_q(2)
    drain_q(1)
    accum_q(1)
    issue_q(3)
    pltpu.make_async_copy(tab.at[:, pl.ds(0, BW)],
                          asm_v.at[pl.ds(0, NS * D)], sem_a).wait()
    h_out = pltpu.async_copy(asm_v.at[pl.ds(0, NS * D)],
                             out.at[pl.ds(0, NS * D), pl.ds(b0, BW)], sem_a)
    drain_q(2)
    accum_q(2)
    drain_q(3)
    accum_q(3, final=True)
    h_pool = pltpu.async_copy(asm_v.at[pl.ds(NS * D, D)],
                              out.at[pl.ds(NS * D, D), pl.ds(b0, BW)], sem_a)
    h_out.wait()
    h_pool.wait()


_sc_call = functools.partial(
    pl.kernel,
    out_type=jax.ShapeDtypeStruct((R, B), jnp.float32),
    mesh=plsc.VectorSubcoreMesh(core_axis_name="c", subcore_axis_name="s"),
    compiler_params=pltpu.CompilerParams(use_tc_tiling_on_sc=False,
                                         skip_device_barrier=True,
                                         disable_bounds_checks=True,
                                         disable_semaphore_checks=True),
    scratch_types=[
        pltpu.VMEM((NS, BW), jnp.int32),
        pltpu.VMEM((L, BW), jnp.int32),
        pltpu.VMEM((R, BW), jnp.float32),
        pltpu.VMEM((D, QL * BW), jnp.float32),
        pltpu.VMEM((D, QL * BW), jnp.float32),
        pltpu.VMEM((D, BW), jnp.float32),
        pltpu.SemaphoreType.DMA,
        pltpu.SemaphoreType.DMA,
    ],
)(_body)


@jax.jit
def kernel(sparse_indices, seq_indices, sparse_tables, seq_table):
    sidxT = jnp.transpose(sparse_indices).astype(jnp.int32)
    qidxT = jnp.transpose(seq_indices).astype(jnp.int32)
    tabT = jnp.transpose(sparse_tables, (0, 2, 1)).reshape(NS * D, V)
    seqT = jnp.transpose(seq_table)

    outT = _sc_call(tabT, seqT, sidxT, qidxT)
    return jnp.transpose(outT.reshape(NF, D, B), (2, 0, 1))


# speedup vs baseline: 1.0114x; 1.0013x over previous
import functools

import jax
import jax.numpy as jnp
from jax import lax
from jax.experimental import pallas as pl
from jax.experimental.pallas import tpu as pltpu
from jax.experimental.pallas import tpu_sc as plsc

B = 4096
NS = 26
V = 100000
D = 16
L = 50
NF = NS + 1
R = NF * D

NC = 2
NSUB = 16
NW = NC * NSUB
BW = B // NW
QL = 13
QS = 12
GD = BW // D


def _body(tab, seq, sidx, qidx, out, sidx_v, qidx_v, asm_v, sq0_v, sq1_v,
          cnt_v, sem_a, sem_b):
    c = lax.axis_index("c")
    s = lax.axis_index("s")
    w = s * NC + c
    b0 = w * BW

    h_sidx = pltpu.async_copy(sidx.at[:, pl.ds(b0, BW)], sidx_v, sem_a)
    h_qidx = pltpu.async_copy(qidx.at[:, pl.ds(b0, BW)], qidx_v, sem_b)
    h_sidx.wait()

    def sparse_f(f, carry):
        for d in range(D):
            pltpu.async_copy(tab.at[f * D + d].at[sidx_v.at[f]],
                             asm_v.at[f * D + d], sem_a)
        return carry

    lax.fori_loop(0, NS, sparse_f, 0)
    h_qidx.wait()

    bufs = [sq0_v, sq1_v, sq0_v, sq1_v]
    offs = [0, QL, 2 * QL, 2 * QL + QS]
    qls = [QL, QL, QS, QS]

    def issue_q(q):
        buf, off = bufs[q], offs[q]

        def body(l, carry):
            for d in range(D):
                pltpu.async_copy(seq.at[d].at[qidx_v.at[off + l]],
                                 buf.at[d, pl.ds(l * BW, BW)], sem_b)
            return carry

        lax.fori_loop(0, qls[q], body, 0)

    def drain_q(q):
        n = qls[q] * BW
        pltpu.make_async_copy(seq.at[:, pl.ds(0, n)],
                              bufs[q].at[:, pl.ds(0, n)], sem_b).wait()

    def clear_acc(k, carry):
        d = k // GD
        g16 = (k % GD) * D
        asm_v[NS * D + d, pl.ds(g16, D)] = jnp.zeros((D,), jnp.float32)
        cnt_v[d, pl.ds(g16, D)] = jnp.zeros((D,), jnp.float32)
        return carry

    def accum_q(q, final=False):
        buf = bufs[q]

        def body(k, carry):
            d = k // GD
            g16 = (k % GD) * D
            acc = asm_v[NS * D + d, pl.ds(g16, D)]
            cnt = cnt_v[d, pl.ds(g16, D)]
            for l in range(qls[q]):
                v = buf[d, pl.ds(l * BW + g16, D)]
                acc = acc + v
                cnt = cnt + jnp.where(v != 0.0, 1.0, 0.0)
            if final:
                asm_v[NS * D + d, pl.ds(g16, D)] = acc / (cnt + 1e-16)
            else:
                asm_v[NS * D + d, pl.ds(g16, D)] = acc
                cnt_v[d, pl.ds(g16, D)] = cnt
            return carry

        lax.fori_loop(0, BW, body, 0)

    issue_q(0)
    issue_q(1)
    lax.fori_loop(0, BW, clear_acc, 0)
    drain_q(0)
    accum_q(0)
    issue_q(2)
    drain_q(1)
    accum_q(1)
    issue_q(3)
    pltpu.make_async_copy(tab.at[:, pl.ds(0, BW)],
                          asm_v.at[pl.ds(0, NS * D)], sem_a).wait()
    h_out = pltpu.async_copy(asm_v.at[pl.ds(0, NS * D)],
                             out.at[pl.ds(0, NS * D), pl.ds(b0, BW)], sem_a)
    drain_q(2)
    accum_q(2)
    drain_q(3)
    accum_q(3, final=True)
    h_pool = pltpu.async_copy(asm_v.at[pl.ds(NS * D, D)],
                              out.at[pl.ds(NS * D, D), pl.ds(b0, BW)], sem_a)
    h_out.wait()
    h_pool.wait()


_sc_call = functools.partial(
    pl.kernel,
    out_type=jax.ShapeDtypeStruct((R, B), jnp.float32),
    mesh=plsc.VectorSubcoreMesh(core_axis_name="c", subcore_axis_name="s"),
    compiler_params=pltpu.CompilerParams(use_tc_tiling_on_sc=False),
    scratch_types=[
        pltpu.VMEM((NS, BW), jnp.int32),
        pltpu.VMEM((L, BW), jnp.int32),
        pltpu.VMEM((R, BW), jnp.float32),
        pltpu.VMEM((D, QL * BW), jnp.float32),
        pltpu.VMEM((D, QL * BW), jnp.float32),
        pltpu.VMEM((D, BW), jnp.float32),
        pltpu.SemaphoreType.DMA,
        pltpu.SemaphoreType.DMA,
    ],
)(_body)


@jax.jit
def kernel(sparse_indices, seq_indices, sparse_tables, seq_table):
    sidxT = jnp.transpose(sparse_indices).astype(jnp.int32)
    qidxT = jnp.transpose(seq_indices).astype(jnp.int32)
    tabT = jnp.transpose(sparse_tables, (0, 2, 1)).reshape(NS * D, V)
    seqT = jnp.transpose(seq_table)

    outT = _sc_call(tabT, seqT, sidxT, qidxT)
    return jnp.transpose(outT.reshape(NF, D, B), (2, 0, 1))
